# baseline (speedup 1.0000x reference)
import functools

import jax
import jax.numpy as jnp
from jax import lax
from jax.experimental import pallas as pl
from jax.experimental.pallas import tpu as pltpu
from jax.experimental.pallas import tpu_sc as plsc

NUM_CORES = 2
NUM_SUBCORES = 16
NUM_WORKERS = NUM_CORES * NUM_SUBCORES
LANES = 16

VOCAB = 1000000
EMB = 64
B = 1024
L = 512
EPS = 1e-6

ROWS = B * L
ROWS_PER_WORKER = ROWS // NUM_WORKERS
BLK = 128
NBLK = ROWS_PER_WORKER // BLK
NBUF = 4

HALF = 512000


def _pair_body(a_ref, b_ref, o_ref):
    o_ref[...] = jnp.concatenate([a_ref[...].T, b_ref[...].T], axis=1)


def _finish_body(x_ref, src_ref, seg_ref, pos_ref, segt_ref, gam_ref, bet_ref,
                 o_ref):
    x = x_ref[...]
    hf = (src_ref[...] >= HALF).astype(jnp.float32)[..., None]
    lo, hi = x[..., :EMB], x[..., EMB:]
    w = lo + hf * (hi - lo)
    st = segt_ref[...]
    sf1 = (seg_ref[...] >= 1).astype(jnp.float32)[..., None]
    sf2 = (seg_ref[...] >= 2).astype(jnp.float32)[..., None]
    seg_e = st[0] + sf1 * (st[1] - st[0]) + sf2 * (st[2] - st[1])
    e = w + pos_ref[...][None] + seg_e
    mean = jnp.mean(e, axis=-1, keepdims=True)
    var = jnp.mean((e - mean) ** 2, axis=-1, keepdims=True)
    normed = (e - mean) * lax.rsqrt(var + EPS)
    out = normed * gam_ref[...] + bet_ref[...]
    o_ref[...] = jnp.transpose(out, (0, 2, 1))


def _sc_body(src_hbm, pairs_hbm, out_hbm, idx_v, rows_v, semw, semo):
    wid = lax.axis_index("s") * NUM_CORES + lax.axis_index("c")
    base = wid * ROWS_PER_WORKER

    def prep(g, b):
        row0 = base + g * BLK
        pltpu.sync_copy(src_hbm.at[pl.ds(row0, BLK)], idx_v.at[b])

        def fix_idx(i, c):
            off = i * LANES
            s = idx_v[b, pl.ds(off, LANES)]
            h = jnp.where(s >= jnp.int32(HALF), jnp.int32(1), jnp.int32(0))
            idx_v[b, pl.ds(off, LANES)] = s - h * jnp.int32(HALF)
            return c
        lax.fori_loop(0, BLK // LANES, fix_idx, 0, unroll=4)

    def w_start(b):
        pltpu.async_copy(pairs_hbm.at[idx_v.at[b]], rows_v.at[b], semw.at[b])

    def w_wait(b):
        pltpu.make_async_copy(
            pairs_hbm.at[idx_v.at[b]], rows_v.at[b], semw.at[b]).wait()

    def o_start(g, b):
        row0 = base + g * BLK
        pltpu.async_copy(
            rows_v.at[b], out_hbm.at[pl.ds(row0, BLK)], semo.at[b])

    def o_wait(g, b):
        row0 = base + g * BLK
        pltpu.make_async_copy(
            rows_v.at[b], out_hbm.at[pl.ds(row0, BLK)], semo.at[b]).wait()

    prep(0, 0)
    w_start(0)

    def outer(go, carry):
        for k in range(NBUF):
            g = go * NBUF + k
            b1 = (k + 1) % NBUF

            @pl.when((g + 1 < NBLK) & (g >= NBUF - 1))
            def _():
                o_wait(g + 1 - NBUF, b1)

            @pl.when(g + 1 < NBLK)
            def _():
                prep(g + 1, b1)
                w_start(b1)

            w_wait(k)
            o_start(g, k)
        return carry

    lax.fori_loop(0, NBLK // NBUF, outer, 0)
    for t in range(NBUF):
        g_last = NBLK - NBUF + t
        o_wait(g_last, g_last % NBUF)


def kernel(src, seg, word_table, position_table, segment_table, ln_gamma, ln_beta):
    src_flat = src.reshape(ROWS).astype(jnp.int32)

    wt_t = word_table.T
    nb = HALF // 4096
    last_b = (VOCAB + 4095) // 4096 - 1
    word_pairs = pl.pallas_call(
        _pair_body,
        grid=(nb,),
        in_specs=[pl.BlockSpec((EMB, 4096), lambda i: (0, i)),
                  pl.BlockSpec((EMB, 4096),
                               lambda i: (0, jnp.minimum(i + nb, last_b)))],
        out_specs=pl.BlockSpec((4096, 2 * EMB), lambda i: (i, 0)),
        out_shape=jax.ShapeDtypeStruct((HALF, 2 * EMB), jnp.float32),
    )(wt_t, wt_t)

    mesh = plsc.VectorSubcoreMesh(
        core_axis_name="c", subcore_axis_name="s",
        num_cores=NUM_CORES, num_subcores=NUM_SUBCORES)

    sc_kernel = functools.partial(
        pl.kernel,
        out_type=jax.ShapeDtypeStruct((ROWS, 2 * EMB), jnp.float32),
        mesh=mesh,
        compiler_params=pltpu.CompilerParams(
            needs_layout_passes=False, use_tc_tiling_on_sc=True),
        scratch_types=[
            pltpu.VMEM((NBUF, BLK), jnp.int32),
            pltpu.VMEM((NBUF, BLK, 2 * EMB), jnp.float32),
            pltpu.SemaphoreType.DMA((NBUF,)),
            pltpu.SemaphoreType.DMA((NBUF,)),
        ],
    )(_sc_body)

    gathered = sc_kernel(src_flat, word_pairs)
    gathered = gathered.reshape(B, L, 2 * EMB)

    out_t = pl.pallas_call(
        _finish_body,
        grid=(B // 8,),
        in_specs=[pl.BlockSpec((8, L, 2 * EMB), lambda i: (i, 0, 0)),
                  pl.BlockSpec((8, L), lambda i: (i, 0)),
                  pl.BlockSpec((8, L), lambda i: (i, 0)),
                  pl.BlockSpec((L, EMB), lambda i: (0, 0)),
                  pl.BlockSpec((3, EMB), lambda i: (0, 0)),
                  pl.BlockSpec((EMB,), lambda i: (0,)),
                  pl.BlockSpec((EMB,), lambda i: (0,))],
        out_specs=pl.BlockSpec((8, EMB, L), lambda i: (i, 0, 0)),
        out_shape=jax.ShapeDtypeStruct((B, EMB, L), jnp.float32),
    )(gathered, src, seg, position_table, segment_table, ln_gamma, ln_beta)
    return jnp.transpose(out_t, (0, 2, 1))
